# baseline (speedup 1.0000x reference)
import jax
import jax.numpy as jnp
import numpy as np
from jax.experimental import pallas as pl
from jax.experimental.pallas import tpu as pltpu

BS, D, H, W = 1, 128, 84, 84
WS, TOPK, NHEAD = 7, 8, 8
M = H // WS
N = W // WS
NW = M * N
WS2 = WS * WS
DIM = D // NHEAD
WSP = 56
SLENP = TOPK * WSP + NW
QSP = NHEAD * WSP
NPADK = TOPK * (WSP - WS2)


def _proj_kernel(x_ref, s_ref, wq_ref, wk_ref, wv_ref,
                 q_ref, k_ref, v_ref, qm_ref, km_ref, vm_ref):
    xw = x_ref[0]
    sw = s_ref[0]
    nt = (((1,), (1,)), ((), ()))
    q = jax.lax.dot_general(xw, wq_ref[...], nt,
                            preferred_element_type=jnp.float32)
    k = jax.lax.dot_general(sw, wk_ref[...], nt,
                            preferred_element_type=jnp.float32)
    v = jax.lax.dot_general(sw, wv_ref[...], nt,
                            preferred_element_type=jnp.float32)
    zpad = jnp.zeros((WSP - WS2, D), jnp.float32)
    q_ref[0] = jnp.concatenate([q, zpad], axis=0)
    k_ref[0] = jnp.concatenate([k, zpad], axis=0)
    v_ref[0] = jnp.concatenate([v, zpad], axis=0)
    qm_ref[0] = jnp.mean(q, axis=0, keepdims=True)
    km_ref[0] = jnp.mean(k, axis=0, keepdims=True)
    vm_ref[0] = jnp.mean(v, axis=0, keepdims=True)


def _topk_kernel(qm_ref, km_ref, idx_ref):
    qm = qm_ref[...].reshape(NW, D)
    km = km_ref[...].reshape(NW, D)
    sim = jax.lax.dot_general(qm, km, (((1,), (1,)), ((), ())),
                              preferred_element_type=jnp.float32)
    col = jax.lax.broadcasted_iota(jnp.int32, (NW, NW), 1)
    neg = jnp.float32(-jnp.inf)
    for j in range(TOPK):
        mx = jnp.max(sim, axis=1, keepdims=True)
        hit = sim >= mx
        cand = jnp.where(hit, col, NW)
        sel = jnp.min(cand, axis=1, keepdims=True)
        idx_ref[:, j:j + 1] = sel
        sim = jnp.where(col == sel, neg, sim)


def _attn_kernel(idx_ref, q_ref, k_ref, v_ref, km_ref, vm_ref, o_ref):
    i = pl.program_id(0)
    qw = q_ref[0]
    lane = jax.lax.broadcasted_iota(jnp.int32, (NHEAD, 1, D), 2)
    head = jax.lax.broadcasted_iota(jnp.int32, (NHEAD, 1, D), 0)
    mask = (lane // DIM == head).astype(jnp.float32)
    scale = jnp.float32(1.0 / np.sqrt(DIM))
    qs = (qw[None, :, :] * (mask * scale)).reshape(QSP, D)

    parts_k = [k_ref[idx_ref[i, j]] for j in range(TOPK)]
    parts_v = [v_ref[idx_ref[i, j]] for j in range(TOPK)]
    kcat = jnp.concatenate(parts_k + [km_ref[...].reshape(NW, D)], axis=0)
    vcat = jnp.concatenate(parts_v + [vm_ref[...].reshape(NW, D)], axis=0)

    s = jax.lax.dot_general(qs, kcat, (((1,), (1,)), ((), ())),
                            preferred_element_type=jnp.float32)
    e = jnp.exp(s)
    denom = jnp.sum(e, axis=1, keepdims=True) - jnp.float32(NPADK)
    msg = jax.lax.dot_general(e, vcat, (((1,), (0,)), ((), ())),
                              preferred_element_type=jnp.float32)
    r = (1.0 / denom).reshape(NHEAD, WSP, 1)
    msg = jnp.sum(msg.reshape(NHEAD, WSP, D) * r * mask, axis=0)
    o_ref[0] = msg[:WS2]


def _layer_norm(v, g, b, eps=1e-5):
    mu = jnp.mean(v, axis=-1, keepdims=True)
    var = jnp.mean((v - mu) ** 2, axis=-1, keepdims=True)
    return (v - mu) / jnp.sqrt(var + eps) * g + b


def _ffn_kernel(xt_ref, msg_ref, wm_ref, fc1a_ref, fc1b_ref, fc1bias_ref,
                dw_ref, dwb_ref, fc2_ref, fc2b_ref, n1w_ref, n1b_ref,
                n2w_ref, n2b_ref, out_ref, pad_ref):
    nt = (((1,), (1,)), ((), ()))
    xt = xt_ref[...]
    merged = jax.lax.dot_general(msg_ref[...], wm_ref[...], nt,
                                 preferred_element_type=jnp.float32)
    merged = _layer_norm(merged, n1w_ref[0], n1b_ref[0])
    y = (jax.lax.dot_general(xt, fc1a_ref[...], nt,
                             preferred_element_type=jnp.float32)
         + jax.lax.dot_general(merged, fc1b_ref[...], nt,
                               preferred_element_type=jnp.float32)
         + fc1bias_ref[0])
    y = jnp.maximum(y, 0.0)
    pad_ref[...] = jnp.zeros_like(pad_ref)
    pad_ref[1:H + 1, 1:W + 1, :] = y.reshape(H, W, 2 * D)
    acc = jnp.zeros((H, W, 2 * D), jnp.float32)
    for di in range(3):
        for dj in range(3):
            acc = acc + pad_ref[di:di + H, dj:dj + W, :] * dw_ref[di * 3 + dj]
    y2 = acc.reshape(H * W, 2 * D) + dwb_ref[0]
    y2 = 0.5 * y2 * (1.0 + jax.lax.erf(y2 * jnp.float32(1.0 / np.sqrt(2.0))))
    z = jax.lax.dot_general(y2, fc2_ref[...], nt,
                            preferred_element_type=jnp.float32) + fc2b_ref[0]
    out_ref[...] = _layer_norm(z, n2w_ref[0], n2b_ref[0]) + xt


@jax.jit
def kernel(x, source, Wq, Wk, Wv, Wm, fc1_w, fc1_b, dw_w, dw_b, fc2_w, fc2_b,
           n1_w, n1_b, n2_w, n2_b):
    f32 = jnp.float32
    xt = jnp.transpose(x, (0, 2, 3, 1)).reshape(H * W, D)
    st = jnp.transpose(source, (0, 2, 3, 1)).reshape(H * W, D)
    xv = xt.reshape(M, WS, N, WS, D).transpose(0, 2, 1, 3, 4).reshape(NW, WS2, D)
    sv = st.reshape(M, WS, N, WS, D).transpose(0, 2, 1, 3, 4).reshape(NW, WS2, D)

    win_in = pl.BlockSpec((1, WS2, D), lambda i: (i, 0, 0))
    full_w = pl.BlockSpec((D, D), lambda i: (0, 0))
    q, k, v, qm, km, vm = pl.pallas_call(
        _proj_kernel,
        grid=(NW,),
        in_specs=[win_in, win_in, full_w, full_w, full_w],
        out_specs=[pl.BlockSpec((1, WSP, D), lambda i: (i, 0, 0)),
                   pl.BlockSpec((1, WSP, D), lambda i: (i, 0, 0)),
                   pl.BlockSpec((1, WSP, D), lambda i: (i, 0, 0)),
                   pl.BlockSpec((1, 1, D), lambda i: (i, 0, 0)),
                   pl.BlockSpec((1, 1, D), lambda i: (i, 0, 0)),
                   pl.BlockSpec((1, 1, D), lambda i: (i, 0, 0))],
        out_shape=[jax.ShapeDtypeStruct((NW, WSP, D), f32)] * 3
                  + [jax.ShapeDtypeStruct((NW, 1, D), f32)] * 3,
    )(xv, sv, Wq, Wk, Wv)

    idx = pl.pallas_call(
        _topk_kernel,
        in_specs=[pl.BlockSpec((NW, 1, D), lambda: (0, 0, 0)),
                  pl.BlockSpec((NW, 1, D), lambda: (0, 0, 0))],
        out_specs=pl.BlockSpec((NW, TOPK), lambda: (0, 0)),
        out_shape=jax.ShapeDtypeStruct((NW, TOPK), jnp.int32),
    )(qm, km)

    msg = pl.pallas_call(
        _attn_kernel,
        grid=(NW,),
        in_specs=[pl.BlockSpec(memory_space=pltpu.SMEM),
                  pl.BlockSpec((1, WSP, D), lambda i: (i, 0, 0)),
                  pl.BlockSpec((NW, WSP, D), lambda i: (0, 0, 0)),
                  pl.BlockSpec((NW, WSP, D), lambda i: (0, 0, 0)),
                  pl.BlockSpec((NW, 1, D), lambda i: (0, 0, 0)),
                  pl.BlockSpec((NW, 1, D), lambda i: (0, 0, 0))],
        out_specs=pl.BlockSpec((1, WS2, D), lambda i: (i, 0, 0)),
        out_shape=jax.ShapeDtypeStruct((NW, WS2, D), f32),
    )(idx, q, k, v, km, vm)
    msg = msg.reshape(H * W, D)

    fc1a = fc1_w[:, :D]
    fc1b = fc1_w[:, D:]
    dwf = jnp.transpose(dw_w[:, 0].reshape(2 * D, 9), (1, 0))
    out2d = pl.pallas_call(
        _ffn_kernel,
        in_specs=[pl.BlockSpec((H * W, D), lambda: (0, 0)),
                  pl.BlockSpec((H * W, D), lambda: (0, 0)),
                  pl.BlockSpec((D, D), lambda: (0, 0)),
                  pl.BlockSpec((2 * D, D), lambda: (0, 0)),
                  pl.BlockSpec((2 * D, D), lambda: (0, 0)),
                  pl.BlockSpec((1, 2 * D), lambda: (0, 0)),
                  pl.BlockSpec((9, 2 * D), lambda: (0, 0)),
                  pl.BlockSpec((1, 2 * D), lambda: (0, 0)),
                  pl.BlockSpec((D, 2 * D), lambda: (0, 0)),
                  pl.BlockSpec((1, D), lambda: (0, 0)),
                  pl.BlockSpec((1, D), lambda: (0, 0)),
                  pl.BlockSpec((1, D), lambda: (0, 0)),
                  pl.BlockSpec((1, D), lambda: (0, 0)),
                  pl.BlockSpec((1, D), lambda: (0, 0))],
        out_specs=pl.BlockSpec((H * W, D), lambda: (0, 0)),
        out_shape=jax.ShapeDtypeStruct((H * W, D), f32),
        scratch_shapes=[pltpu.VMEM((H + 2, W + 2, 2 * D), f32)],
    )(xt, msg, Wm, fc1a, fc1b, fc1_b[None, :], dwf, dw_b[None, :], fc2_w,
      fc2_b[None, :], n1_w[None, :], n1_b[None, :], n2_w[None, :],
      n2_b[None, :])

    out = jnp.transpose(out2d.reshape(1, H, W, D), (0, 3, 1, 2))
    return out


# speedup vs baseline: 1.5701x; 1.5701x over previous
import jax
import jax.numpy as jnp
import numpy as np
from jax.experimental import pallas as pl
from jax.experimental.pallas import tpu as pltpu

BS, D, H, W = 1, 128, 84, 84
WS, TOPK, NHEAD = 7, 8, 8
M = H // WS
N = W // WS
NW = M * N
WS2 = WS * WS
DIM = D // NHEAD
WSP = 56
SLENP = TOPK * WSP + NW
QSP = NHEAD * WSP
NPADK = TOPK * (WSP - WS2)


PROJ_B = 8


def _proj_kernel(x_ref, s_ref, wq_ref, wk_ref, wv_ref,
                 q_ref, k_ref, v_ref, qm_ref, km_ref, vm_ref):
    xw = x_ref[...].reshape(PROJ_B * WSP, D)
    sw = s_ref[...].reshape(PROJ_B * WSP, D)
    nt = (((1,), (1,)), ((), ()))
    rcp = jnp.float32(1.0 / WS2)
    q = jax.lax.dot_general(xw, wq_ref[...], nt,
                            preferred_element_type=jnp.float32)
    k = jax.lax.dot_general(sw, wk_ref[...], nt,
                            preferred_element_type=jnp.float32)
    v = jax.lax.dot_general(sw, wv_ref[...], nt,
                            preferred_element_type=jnp.float32)
    q = q.reshape(PROJ_B, WSP, D)
    k = k.reshape(PROJ_B, WSP, D)
    v = v.reshape(PROJ_B, WSP, D)
    q_ref[...] = q
    k_ref[...] = k
    v_ref[...] = v
    qm_ref[...] = jnp.sum(q, axis=1, keepdims=True) * rcp
    km_ref[...] = jnp.sum(k, axis=1, keepdims=True) * rcp
    vm_ref[...] = jnp.sum(v, axis=1, keepdims=True) * rcp


def _topk_kernel(qm_ref, km_ref, idx_ref):
    qm = qm_ref[...].reshape(NW, D)
    km = km_ref[...].reshape(NW, D)
    sim = jax.lax.dot_general(qm, km, (((1,), (1,)), ((), ())),
                              preferred_element_type=jnp.float32)
    col = jax.lax.broadcasted_iota(jnp.int32, (NW, NW), 1)
    neg = jnp.float32(-jnp.inf)
    for j in range(TOPK):
        mx = jnp.max(sim, axis=1, keepdims=True)
        hit = sim >= mx
        cand = jnp.where(hit, col, NW)
        sel = jnp.min(cand, axis=1, keepdims=True)
        idx_ref[:, j:j + 1] = sel
        sim = jnp.where(col == sel, neg, sim)


ATTN_B = 4


def _attn_kernel(idx_ref, q_ref, k_ref, v_ref, km_ref, vm_ref, o_ref):
    i = pl.program_id(0)
    lane = jax.lax.broadcasted_iota(jnp.int32, (NHEAD, 1, D), 2)
    head = jax.lax.broadcasted_iota(jnp.int32, (NHEAD, 1, D), 0)
    mask = (lane // DIM == head).astype(jnp.float32)
    qmask = mask * jnp.float32(1.0 / np.sqrt(DIM))
    kmr = km_ref[...].reshape(NW, D)
    vmr = vm_ref[...].reshape(NW, D)
    for b in range(ATTN_B):
        w = i * ATTN_B + b
        qw = q_ref[b]
        qs = (qw[None, :, :] * qmask).reshape(QSP, D)
        parts_k = [k_ref[idx_ref[w, j]] for j in range(TOPK)]
        parts_v = [v_ref[idx_ref[w, j]] for j in range(TOPK)]
        kcat = jnp.concatenate(parts_k + [kmr], axis=0)
        vcat = jnp.concatenate(parts_v + [vmr], axis=0)

        s = jax.lax.dot_general(qs, kcat, (((1,), (1,)), ((), ())),
                                preferred_element_type=jnp.float32)
        e = jnp.exp(s)
        denom = jnp.sum(e, axis=1, keepdims=True) - jnp.float32(NPADK)
        msg = jax.lax.dot_general(e, vcat, (((1,), (0,)), ((), ())),
                                  preferred_element_type=jnp.float32)
        r = (1.0 / denom).reshape(NHEAD, WSP, 1)
        msg = jnp.sum(msg.reshape(NHEAD, WSP, D) * r * mask, axis=0)
        o_ref[b] = msg[:WS2]


def _layer_norm(v, g, b, eps=1e-5):
    mu = jnp.mean(v, axis=-1, keepdims=True)
    var = jnp.mean((v - mu) ** 2, axis=-1, keepdims=True)
    return (v - mu) / jnp.sqrt(var + eps) * g + b


def _ffn_kernel(xt_ref, msg_ref, wm_ref, fc1a_ref, fc1b_ref, fc1bias_ref,
                dw_ref, dwb_ref, fc2_ref, fc2b_ref, n1w_ref, n1b_ref,
                n2w_ref, n2b_ref, out_ref, pad_ref):
    nt = (((1,), (1,)), ((), ()))
    xt = xt_ref[...]
    merged = jax.lax.dot_general(msg_ref[...], wm_ref[...], nt,
                                 preferred_element_type=jnp.float32)
    merged = _layer_norm(merged, n1w_ref[0], n1b_ref[0])
    y = (jax.lax.dot_general(xt, fc1a_ref[...], nt,
                             preferred_element_type=jnp.float32)
         + jax.lax.dot_general(merged, fc1b_ref[...], nt,
                               preferred_element_type=jnp.float32)
         + fc1bias_ref[0])
    y = jnp.maximum(y, 0.0)
    pad_ref[...] = jnp.zeros_like(pad_ref)
    pad_ref[1:H + 1, 1:W + 1, :] = y.reshape(H, W, 2 * D)
    acc = jnp.zeros((H, W, 2 * D), jnp.float32)
    for di in range(3):
        for dj in range(3):
            acc = acc + pad_ref[di:di + H, dj:dj + W, :] * dw_ref[di * 3 + dj]
    y2 = acc.reshape(H * W, 2 * D) + dwb_ref[0]
    y2 = 0.5 * y2 * (1.0 + jax.lax.erf(y2 * jnp.float32(1.0 / np.sqrt(2.0))))
    z = jax.lax.dot_general(y2, fc2_ref[...], nt,
                            preferred_element_type=jnp.float32) + fc2b_ref[0]
    out_ref[...] = _layer_norm(z, n2w_ref[0], n2b_ref[0]) + xt


@jax.jit
def kernel(x, source, Wq, Wk, Wv, Wm, fc1_w, fc1_b, dw_w, dw_b, fc2_w, fc2_b,
           n1_w, n1_b, n2_w, n2_b):
    f32 = jnp.float32
    xt = jnp.transpose(x, (0, 2, 3, 1)).reshape(H * W, D)
    st = jnp.transpose(source, (0, 2, 3, 1)).reshape(H * W, D)
    xv = jnp.pad(
        xt.reshape(M, WS, N, WS, D).transpose(0, 2, 1, 3, 4).reshape(NW, WS2, D),
        ((0, 0), (0, WSP - WS2), (0, 0)))
    sv = jnp.pad(
        st.reshape(M, WS, N, WS, D).transpose(0, 2, 1, 3, 4).reshape(NW, WS2, D),
        ((0, 0), (0, WSP - WS2), (0, 0)))

    win_in = pl.BlockSpec((PROJ_B, WSP, D), lambda i: (i, 0, 0))
    full_w = pl.BlockSpec((D, D), lambda i: (0, 0))
    q, k, v, qm, km, vm = pl.pallas_call(
        _proj_kernel,
        grid=(NW // PROJ_B,),
        in_specs=[win_in, win_in, full_w, full_w, full_w],
        out_specs=[pl.BlockSpec((PROJ_B, WSP, D), lambda i: (i, 0, 0)),
                   pl.BlockSpec((PROJ_B, WSP, D), lambda i: (i, 0, 0)),
                   pl.BlockSpec((PROJ_B, WSP, D), lambda i: (i, 0, 0)),
                   pl.BlockSpec((PROJ_B, 1, D), lambda i: (i, 0, 0)),
                   pl.BlockSpec((PROJ_B, 1, D), lambda i: (i, 0, 0)),
                   pl.BlockSpec((PROJ_B, 1, D), lambda i: (i, 0, 0))],
        out_shape=[jax.ShapeDtypeStruct((NW, WSP, D), f32)] * 3
                  + [jax.ShapeDtypeStruct((NW, 1, D), f32)] * 3,
    )(xv, sv, Wq, Wk, Wv)

    idx = pl.pallas_call(
        _topk_kernel,
        in_specs=[pl.BlockSpec((NW, 1, D), lambda: (0, 0, 0)),
                  pl.BlockSpec((NW, 1, D), lambda: (0, 0, 0))],
        out_specs=pl.BlockSpec((NW, TOPK), lambda: (0, 0)),
        out_shape=jax.ShapeDtypeStruct((NW, TOPK), jnp.int32),
    )(qm, km)

    msg = pl.pallas_call(
        _attn_kernel,
        grid=(NW // ATTN_B,),
        in_specs=[pl.BlockSpec(memory_space=pltpu.SMEM),
                  pl.BlockSpec((ATTN_B, WSP, D), lambda i: (i, 0, 0)),
                  pl.BlockSpec((NW, WSP, D), lambda i: (0, 0, 0)),
                  pl.BlockSpec((NW, WSP, D), lambda i: (0, 0, 0)),
                  pl.BlockSpec((NW, 1, D), lambda i: (0, 0, 0)),
                  pl.BlockSpec((NW, 1, D), lambda i: (0, 0, 0))],
        out_specs=pl.BlockSpec((ATTN_B, WS2, D), lambda i: (i, 0, 0)),
        out_shape=jax.ShapeDtypeStruct((NW, WS2, D), f32),
    )(idx, q, k, v, km, vm)
    msg = msg.reshape(H * W, D)

    fc1a = fc1_w[:, :D]
    fc1b = fc1_w[:, D:]
    dwf = jnp.transpose(dw_w[:, 0].reshape(2 * D, 9), (1, 0))
    out2d = pl.pallas_call(
        _ffn_kernel,
        in_specs=[pl.BlockSpec((H * W, D), lambda: (0, 0)),
                  pl.BlockSpec((H * W, D), lambda: (0, 0)),
                  pl.BlockSpec((D, D), lambda: (0, 0)),
                  pl.BlockSpec((2 * D, D), lambda: (0, 0)),
                  pl.BlockSpec((2 * D, D), lambda: (0, 0)),
                  pl.BlockSpec((1, 2 * D), lambda: (0, 0)),
                  pl.BlockSpec((9, 2 * D), lambda: (0, 0)),
                  pl.BlockSpec((1, 2 * D), lambda: (0, 0)),
                  pl.BlockSpec((D, 2 * D), lambda: (0, 0)),
                  pl.BlockSpec((1, D), lambda: (0, 0)),
                  pl.BlockSpec((1, D), lambda: (0, 0)),
                  pl.BlockSpec((1, D), lambda: (0, 0)),
                  pl.BlockSpec((1, D), lambda: (0, 0)),
                  pl.BlockSpec((1, D), lambda: (0, 0))],
        out_specs=pl.BlockSpec((H * W, D), lambda: (0, 0)),
        out_shape=jax.ShapeDtypeStruct((H * W, D), f32),
        scratch_shapes=[pltpu.VMEM((H + 2, W + 2, 2 * D), f32)],
    )(xt, msg, Wm, fc1a, fc1b, fc1_b[None, :], dwf, dw_b[None, :], fc2_w,
      fc2_b[None, :], n1_w[None, :], n1_b[None, :], n2_w[None, :],
      n2_b[None, :])

    out = jnp.transpose(out2d.reshape(1, H, W, D), (0, 3, 1, 2))
    return out


# speedup vs baseline: 1.7843x; 1.1364x over previous
import jax
import jax.numpy as jnp
import numpy as np
from jax.experimental import pallas as pl
from jax.experimental.pallas import tpu as pltpu

BS, D, H, W = 1, 128, 84, 84
WS, TOPK, NHEAD = 7, 8, 8
M = H // WS
N = W // WS
NW = M * N
WS2 = WS * WS
DIM = D // NHEAD
WSP = 56
SLENP = TOPK * WSP + NW
QSP = NHEAD * WSP
NPADK = TOPK * (WSP - WS2)


PROJ_B = 8


def _proj_kernel(x_ref, s_ref, wq_ref, wk_ref, wv_ref,
                 q_ref, k_ref, v_ref, qm_ref, km_ref, vm_ref):
    xw = x_ref[...].reshape(PROJ_B * WSP, D)
    sw = s_ref[...].reshape(PROJ_B * WSP, D)
    nt = (((1,), (1,)), ((), ()))
    rcp = jnp.float32(1.0 / WS2)
    q = jax.lax.dot_general(xw, wq_ref[...], nt,
                            preferred_element_type=jnp.float32)
    k = jax.lax.dot_general(sw, wk_ref[...], nt,
                            preferred_element_type=jnp.float32)
    v = jax.lax.dot_general(sw, wv_ref[...], nt,
                            preferred_element_type=jnp.float32)
    q = q.reshape(PROJ_B, WSP, D)
    k = k.reshape(PROJ_B, WSP, D)
    v = v.reshape(PROJ_B, WSP, D)
    q_ref[...] = q
    k_ref[...] = k
    v_ref[...] = v
    qm_ref[...] = jnp.sum(q, axis=1, keepdims=True) * rcp
    km_ref[...] = jnp.sum(k, axis=1, keepdims=True) * rcp
    vm_ref[...] = jnp.sum(v, axis=1, keepdims=True) * rcp


def _topk_kernel(qm_ref, km_ref, idx_ref):
    qm = qm_ref[...].reshape(NW, D)
    km = km_ref[...].reshape(NW, D)
    sim = jax.lax.dot_general(qm, km, (((1,), (1,)), ((), ())),
                              preferred_element_type=jnp.float32)
    col = jax.lax.broadcasted_iota(jnp.int32, (NW, NW), 1)
    neg = jnp.float32(-jnp.inf)
    for j in range(TOPK):
        mx = jnp.max(sim, axis=1, keepdims=True)
        hit = sim >= mx
        cand = jnp.where(hit, col, NW)
        sel = jnp.min(cand, axis=1, keepdims=True)
        idx_ref[:, j:j + 1] = sel
        sim = jnp.where(col == sel, neg, sim)


ATTN_B = 4


def _attn_kernel(idx_ref, q_ref, k_ref, v_ref, km_ref, vm_ref, o_ref):
    i = pl.program_id(0)
    lane = jax.lax.broadcasted_iota(jnp.int32, (NHEAD, 1, D), 2)
    head = jax.lax.broadcasted_iota(jnp.int32, (NHEAD, 1, D), 0)
    mask = (lane // DIM == head).astype(jnp.float32)
    qmask = mask * jnp.float32(1.0 / np.sqrt(DIM))
    kmr = km_ref[...].reshape(NW, D)
    vmr = vm_ref[...].reshape(NW, D)
    for b in range(ATTN_B):
        w = i * ATTN_B + b
        qw = q_ref[b]
        qs = (qw[None, :, :] * qmask).reshape(QSP, D)
        parts_k = [k_ref[idx_ref[w, j]] for j in range(TOPK)]
        parts_v = [v_ref[idx_ref[w, j]] for j in range(TOPK)]
        kcat = jnp.concatenate(parts_k + [kmr], axis=0)
        vcat = jnp.concatenate(parts_v + [vmr], axis=0)

        s = jax.lax.dot_general(qs, kcat, (((1,), (1,)), ((), ())),
                                preferred_element_type=jnp.float32)
        e = jnp.exp(s)
        denom = jnp.sum(e, axis=1, keepdims=True) - jnp.float32(NPADK)
        msg = jax.lax.dot_general(e, vcat, (((1,), (0,)), ((), ())),
                                  preferred_element_type=jnp.float32)
        r = (1.0 / denom).reshape(NHEAD, WSP, 1)
        msg = jnp.sum(msg.reshape(NHEAD, WSP, D) * r * mask, axis=0)
        o_ref[b] = msg[:WS2]


def _layer_norm(v, g, b, eps=1e-5):
    mu = jnp.mean(v, axis=-1, keepdims=True)
    var = jnp.mean((v - mu) ** 2, axis=-1, keepdims=True)
    return (v - mu) / jnp.sqrt(var + eps) * g + b


def _ffn_kernel(xt_ref, msg_ref, wm_ref, fc1a_ref, fc1b_ref, fc1bias_ref,
                dw_ref, dwb_ref, fc2_ref, fc2b_ref, n1w_ref, n1b_ref,
                n2w_ref, n2b_ref, out_ref, pad_ref):
    nt = (((1,), (1,)), ((), ()))
    xt = xt_ref[...]
    merged = jax.lax.dot_general(msg_ref[...], wm_ref[...], nt,
                                 preferred_element_type=jnp.float32)
    merged = _layer_norm(merged, n1w_ref[0], n1b_ref[0])
    y = (jax.lax.dot_general(xt, fc1a_ref[...], nt,
                             preferred_element_type=jnp.float32)
         + jax.lax.dot_general(merged, fc1b_ref[...], nt,
                               preferred_element_type=jnp.float32)
         + fc1bias_ref[0])
    y = jnp.maximum(y, 0.0)
    pad_ref[...] = jnp.zeros_like(pad_ref)
    pad_ref[1:H + 1, 1:W + 1, :] = y.reshape(H, W, 2 * D)
    acc = jnp.zeros((H, W, 2 * D), jnp.float32)
    for di in range(3):
        for dj in range(3):
            acc = acc + pad_ref[di:di + H, dj:dj + W, :] * dw_ref[di * 3 + dj]
    y2 = acc.reshape(H * W, 2 * D) + dwb_ref[0]
    y2 = 0.5 * y2 * (1.0 + jax.lax.erf(y2 * jnp.float32(1.0 / np.sqrt(2.0))))
    z = jax.lax.dot_general(y2, fc2_ref[...], nt,
                            preferred_element_type=jnp.float32) + fc2b_ref[0]
    out_ref[...] = _layer_norm(z, n2w_ref[0], n2b_ref[0]) + xt


@jax.jit
def kernel(x, source, Wq, Wk, Wv, Wm, fc1_w, fc1_b, dw_w, dw_b, fc2_w, fc2_b,
           n1_w, n1_b, n2_w, n2_b):
    f32 = jnp.float32
    xt = jnp.transpose(x, (0, 2, 3, 1)).reshape(H * W, D)
    xv = jnp.pad(
        jnp.transpose(x.reshape(D, M, WS, N, WS), (1, 3, 2, 4, 0)
                      ).reshape(NW, WS2, D),
        ((0, 0), (0, WSP - WS2), (0, 0)))
    sv = jnp.pad(
        jnp.transpose(source.reshape(D, M, WS, N, WS), (1, 3, 2, 4, 0)
                      ).reshape(NW, WS2, D),
        ((0, 0), (0, WSP - WS2), (0, 0)))

    win_in = pl.BlockSpec((PROJ_B, WSP, D), lambda i: (i, 0, 0))
    full_w = pl.BlockSpec((D, D), lambda i: (0, 0))
    q, k, v, qm, km, vm = pl.pallas_call(
        _proj_kernel,
        grid=(NW // PROJ_B,),
        in_specs=[win_in, win_in, full_w, full_w, full_w],
        out_specs=[pl.BlockSpec((PROJ_B, WSP, D), lambda i: (i, 0, 0)),
                   pl.BlockSpec((PROJ_B, WSP, D), lambda i: (i, 0, 0)),
                   pl.BlockSpec((PROJ_B, WSP, D), lambda i: (i, 0, 0)),
                   pl.BlockSpec((PROJ_B, 1, D), lambda i: (i, 0, 0)),
                   pl.BlockSpec((PROJ_B, 1, D), lambda i: (i, 0, 0)),
                   pl.BlockSpec((PROJ_B, 1, D), lambda i: (i, 0, 0))],
        out_shape=[jax.ShapeDtypeStruct((NW, WSP, D), f32)] * 3
                  + [jax.ShapeDtypeStruct((NW, 1, D), f32)] * 3,
    )(xv, sv, Wq, Wk, Wv)

    idx = pl.pallas_call(
        _topk_kernel,
        in_specs=[pl.BlockSpec((NW, 1, D), lambda: (0, 0, 0)),
                  pl.BlockSpec((NW, 1, D), lambda: (0, 0, 0))],
        out_specs=pl.BlockSpec((NW, TOPK), lambda: (0, 0)),
        out_shape=jax.ShapeDtypeStruct((NW, TOPK), jnp.int32),
    )(qm, km)

    msg = pl.pallas_call(
        _attn_kernel,
        grid=(NW // ATTN_B,),
        in_specs=[pl.BlockSpec(memory_space=pltpu.SMEM),
                  pl.BlockSpec((ATTN_B, WSP, D), lambda i: (i, 0, 0)),
                  pl.BlockSpec((NW, WSP, D), lambda i: (0, 0, 0)),
                  pl.BlockSpec((NW, WSP, D), lambda i: (0, 0, 0)),
                  pl.BlockSpec((NW, 1, D), lambda i: (0, 0, 0)),
                  pl.BlockSpec((NW, 1, D), lambda i: (0, 0, 0))],
        out_specs=pl.BlockSpec((ATTN_B, WS2, D), lambda i: (i, 0, 0)),
        out_shape=jax.ShapeDtypeStruct((NW, WS2, D), f32),
    )(idx, q, k, v, km, vm)
    msg = msg.reshape(H * W, D)

    fc1a = fc1_w[:, :D]
    fc1b = fc1_w[:, D:]
    dwf = jnp.transpose(dw_w[:, 0].reshape(2 * D, 9), (1, 0))
    out2d = pl.pallas_call(
        _ffn_kernel,
        in_specs=[pl.BlockSpec((H * W, D), lambda: (0, 0)),
                  pl.BlockSpec((H * W, D), lambda: (0, 0)),
                  pl.BlockSpec((D, D), lambda: (0, 0)),
                  pl.BlockSpec((2 * D, D), lambda: (0, 0)),
                  pl.BlockSpec((2 * D, D), lambda: (0, 0)),
                  pl.BlockSpec((1, 2 * D), lambda: (0, 0)),
                  pl.BlockSpec((9, 2 * D), lambda: (0, 0)),
                  pl.BlockSpec((1, 2 * D), lambda: (0, 0)),
                  pl.BlockSpec((D, 2 * D), lambda: (0, 0)),
                  pl.BlockSpec((1, D), lambda: (0, 0)),
                  pl.BlockSpec((1, D), lambda: (0, 0)),
                  pl.BlockSpec((1, D), lambda: (0, 0)),
                  pl.BlockSpec((1, D), lambda: (0, 0)),
                  pl.BlockSpec((1, D), lambda: (0, 0))],
        out_specs=pl.BlockSpec((H * W, D), lambda: (0, 0)),
        out_shape=jax.ShapeDtypeStruct((H * W, D), f32),
        scratch_shapes=[pltpu.VMEM((H + 2, W + 2, 2 * D), f32)],
    )(xt, msg, Wm, fc1a, fc1b, fc1_b[None, :], dwf, dw_b[None, :], fc2_w,
      fc2_b[None, :], n1_w[None, :], n1_b[None, :], n2_w[None, :],
      n2_b[None, :])

    out = jnp.transpose(out2d.reshape(1, H, W, D), (0, 3, 1, 2))
    return out


# speedup vs baseline: 1.9084x; 1.0695x over previous
import jax
import jax.numpy as jnp
import numpy as np
from jax.experimental import pallas as pl
from jax.experimental.pallas import tpu as pltpu

BS, D, H, W = 1, 128, 84, 84
WS, TOPK, NHEAD = 7, 8, 8
M = H // WS
N = W // WS
NW = M * N
WS2 = WS * WS
DIM = D // NHEAD
WSP = 56
SLENP = TOPK * WSP + NW
QSP = NHEAD * WSP
NPADK = TOPK * (WSP - WS2)


PROJ_B = 16


def _proj_kernel(x_ref, s_ref, wq_ref, wk_ref, wv_ref,
                 q_ref, k_ref, v_ref, qm_ref, km_ref, vm_ref):
    xw = x_ref[...].reshape(PROJ_B * WSP, D)
    sw = s_ref[...].reshape(PROJ_B * WSP, D)
    nt = (((1,), (1,)), ((), ()))
    rcp = jnp.float32(1.0 / WS2)
    q = jax.lax.dot_general(xw, wq_ref[...], nt,
                            preferred_element_type=jnp.float32)
    k = jax.lax.dot_general(sw, wk_ref[...], nt,
                            preferred_element_type=jnp.float32)
    v = jax.lax.dot_general(sw, wv_ref[...], nt,
                            preferred_element_type=jnp.float32)
    q = q.reshape(PROJ_B, WSP, D)
    k = k.reshape(PROJ_B, WSP, D)
    v = v.reshape(PROJ_B, WSP, D)
    q_ref[...] = q
    k_ref[...] = k
    v_ref[...] = v
    qm_ref[...] = jnp.sum(q, axis=1, keepdims=True) * rcp
    km_ref[...] = jnp.sum(k, axis=1, keepdims=True) * rcp
    vm_ref[...] = jnp.sum(v, axis=1, keepdims=True) * rcp


def _topk_kernel(qm_ref, km_ref, idx_ref):
    qm = qm_ref[...].reshape(NW, D)
    km = km_ref[...].reshape(NW, D)
    sim = jax.lax.dot_general(qm, km, (((1,), (1,)), ((), ())),
                              preferred_element_type=jnp.float32)
    col = jax.lax.broadcasted_iota(jnp.int32, (NW, NW), 1)
    neg = jnp.float32(-jnp.inf)
    for j in range(TOPK):
        mx = jnp.max(sim, axis=1, keepdims=True)
        hit = sim >= mx
        cand = jnp.where(hit, col, NW)
        sel = jnp.min(cand, axis=1, keepdims=True)
        idx_ref[:, j:j + 1] = sel
        sim = jnp.where(col == sel, neg, sim)


ATTN_B = 8


def _attn_kernel(idx_ref, q_ref, k_ref, v_ref, km_ref, vm_ref, o_ref):
    i = pl.program_id(0)
    lane = jax.lax.broadcasted_iota(jnp.int32, (NHEAD, 1, D), 2)
    head = jax.lax.broadcasted_iota(jnp.int32, (NHEAD, 1, D), 0)
    mask = (lane // DIM == head).astype(jnp.float32)
    qmask = mask * jnp.float32(1.0 / np.sqrt(DIM))
    kmr = km_ref[...].reshape(NW, D)
    vmr = vm_ref[...].reshape(NW, D)
    for b in range(ATTN_B):
        w = i * ATTN_B + b
        qw = q_ref[b]
        qs = (qw[None, :, :] * qmask).reshape(QSP, D)
        parts_k = [k_ref[idx_ref[w, j]] for j in range(TOPK)]
        parts_v = [v_ref[idx_ref[w, j]] for j in range(TOPK)]
        kcat = jnp.concatenate(parts_k + [kmr], axis=0)
        vcat = jnp.concatenate(parts_v + [vmr], axis=0)

        s = jax.lax.dot_general(qs, kcat, (((1,), (1,)), ((), ())),
                                preferred_element_type=jnp.float32)
        e = jnp.exp(s)
        denom = jnp.sum(e, axis=1, keepdims=True) - jnp.float32(NPADK)
        msg = jax.lax.dot_general(e, vcat, (((1,), (0,)), ((), ())),
                                  preferred_element_type=jnp.float32)
        r = (1.0 / denom).reshape(NHEAD, WSP, 1)
        msg = jnp.sum(msg.reshape(NHEAD, WSP, D) * r * mask, axis=0)
        o_ref[b] = msg[:WS2]


def _layer_norm(v, g, b, eps=1e-5):
    mu = jnp.mean(v, axis=-1, keepdims=True)
    var = jnp.mean((v - mu) ** 2, axis=-1, keepdims=True)
    return (v - mu) / jnp.sqrt(var + eps) * g + b


def _ffn_kernel(xt_ref, msg_ref, wm_ref, fc1a_ref, fc1b_ref, fc1bias_ref,
                dw_ref, dwb_ref, fc2_ref, fc2b_ref, n1w_ref, n1b_ref,
                n2w_ref, n2b_ref, out_ref, pad_ref):
    nt = (((1,), (1,)), ((), ()))
    xt = xt_ref[...]
    merged = jax.lax.dot_general(msg_ref[...], wm_ref[...], nt,
                                 preferred_element_type=jnp.float32)
    merged = _layer_norm(merged, n1w_ref[0], n1b_ref[0])
    y = (jax.lax.dot_general(xt, fc1a_ref[...], nt,
                             preferred_element_type=jnp.float32)
         + jax.lax.dot_general(merged, fc1b_ref[...], nt,
                               preferred_element_type=jnp.float32)
         + fc1bias_ref[0])
    y = jnp.maximum(y, 0.0)
    pad_ref[...] = jnp.zeros_like(pad_ref)
    pad_ref[1:H + 1, 1:W + 1, :] = y.reshape(H, W, 2 * D)
    acc = jnp.zeros((H, W, 2 * D), jnp.float32)
    for di in range(3):
        for dj in range(3):
            acc = acc + pad_ref[di:di + H, dj:dj + W, :] * dw_ref[di * 3 + dj]
    y2 = acc.reshape(H * W, 2 * D) + dwb_ref[0]
    y2 = 0.5 * y2 * (1.0 + jax.lax.erf(y2 * jnp.float32(1.0 / np.sqrt(2.0))))
    z = jax.lax.dot_general(y2, fc2_ref[...], nt,
                            preferred_element_type=jnp.float32) + fc2b_ref[0]
    out_ref[...] = _layer_norm(z, n2w_ref[0], n2b_ref[0]) + xt


@jax.jit
def kernel(x, source, Wq, Wk, Wv, Wm, fc1_w, fc1_b, dw_w, dw_b, fc2_w, fc2_b,
           n1_w, n1_b, n2_w, n2_b):
    f32 = jnp.float32
    xt = jnp.transpose(x, (0, 2, 3, 1)).reshape(H * W, D)
    xv = jnp.pad(
        jnp.transpose(x.reshape(D, M, WS, N, WS), (1, 3, 2, 4, 0)
                      ).reshape(NW, WS2, D),
        ((0, 0), (0, WSP - WS2), (0, 0)))
    sv = jnp.pad(
        jnp.transpose(source.reshape(D, M, WS, N, WS), (1, 3, 2, 4, 0)
                      ).reshape(NW, WS2, D),
        ((0, 0), (0, WSP - WS2), (0, 0)))

    win_in = pl.BlockSpec((PROJ_B, WSP, D), lambda i: (i, 0, 0))
    full_w = pl.BlockSpec((D, D), lambda i: (0, 0))
    q, k, v, qm, km, vm = pl.pallas_call(
        _proj_kernel,
        grid=(NW // PROJ_B,),
        in_specs=[win_in, win_in, full_w, full_w, full_w],
        out_specs=[pl.BlockSpec((PROJ_B, WSP, D), lambda i: (i, 0, 0)),
                   pl.BlockSpec((PROJ_B, WSP, D), lambda i: (i, 0, 0)),
                   pl.BlockSpec((PROJ_B, WSP, D), lambda i: (i, 0, 0)),
                   pl.BlockSpec((PROJ_B, 1, D), lambda i: (i, 0, 0)),
                   pl.BlockSpec((PROJ_B, 1, D), lambda i: (i, 0, 0)),
                   pl.BlockSpec((PROJ_B, 1, D), lambda i: (i, 0, 0))],
        out_shape=[jax.ShapeDtypeStruct((NW, WSP, D), f32)] * 3
                  + [jax.ShapeDtypeStruct((NW, 1, D), f32)] * 3,
    )(xv, sv, Wq, Wk, Wv)

    idx = pl.pallas_call(
        _topk_kernel,
        in_specs=[pl.BlockSpec((NW, 1, D), lambda: (0, 0, 0)),
                  pl.BlockSpec((NW, 1, D), lambda: (0, 0, 0))],
        out_specs=pl.BlockSpec((NW, TOPK), lambda: (0, 0)),
        out_shape=jax.ShapeDtypeStruct((NW, TOPK), jnp.int32),
    )(qm, km)

    msg = pl.pallas_call(
        _attn_kernel,
        grid=(NW // ATTN_B,),
        in_specs=[pl.BlockSpec(memory_space=pltpu.SMEM),
                  pl.BlockSpec((ATTN_B, WSP, D), lambda i: (i, 0, 0)),
                  pl.BlockSpec((NW, WSP, D), lambda i: (0, 0, 0)),
                  pl.BlockSpec((NW, WSP, D), lambda i: (0, 0, 0)),
                  pl.BlockSpec((NW, 1, D), lambda i: (0, 0, 0)),
                  pl.BlockSpec((NW, 1, D), lambda i: (0, 0, 0))],
        out_specs=pl.BlockSpec((ATTN_B, WS2, D), lambda i: (i, 0, 0)),
        out_shape=jax.ShapeDtypeStruct((NW, WS2, D), f32),
    )(idx, q, k, v, km, vm)
    msg = msg.reshape(H * W, D)

    fc1a = fc1_w[:, :D]
    fc1b = fc1_w[:, D:]
    dwf = jnp.transpose(dw_w[:, 0].reshape(2 * D, 9), (1, 0))
    out2d = pl.pallas_call(
        _ffn_kernel,
        in_specs=[pl.BlockSpec((H * W, D), lambda: (0, 0)),
                  pl.BlockSpec((H * W, D), lambda: (0, 0)),
                  pl.BlockSpec((D, D), lambda: (0, 0)),
                  pl.BlockSpec((2 * D, D), lambda: (0, 0)),
                  pl.BlockSpec((2 * D, D), lambda: (0, 0)),
                  pl.BlockSpec((1, 2 * D), lambda: (0, 0)),
                  pl.BlockSpec((9, 2 * D), lambda: (0, 0)),
                  pl.BlockSpec((1, 2 * D), lambda: (0, 0)),
                  pl.BlockSpec((D, 2 * D), lambda: (0, 0)),
                  pl.BlockSpec((1, D), lambda: (0, 0)),
                  pl.BlockSpec((1, D), lambda: (0, 0)),
                  pl.BlockSpec((1, D), lambda: (0, 0)),
                  pl.BlockSpec((1, D), lambda: (0, 0)),
                  pl.BlockSpec((1, D), lambda: (0, 0))],
        out_specs=pl.BlockSpec((H * W, D), lambda: (0, 0)),
        out_shape=jax.ShapeDtypeStruct((H * W, D), f32),
        scratch_shapes=[pltpu.VMEM((H + 2, W + 2, 2 * D), f32)],
    )(xt, msg, Wm, fc1a, fc1b, fc1_b[None, :], dwf, dw_b[None, :], fc2_w,
      fc2_b[None, :], n1_w[None, :], n1_b[None, :], n2_w[None, :],
      n2_b[None, :])

    out = jnp.transpose(out2d.reshape(1, H, W, D), (0, 3, 1, 2))
    return out


# speedup vs baseline: 1.9932x; 1.0444x over previous
import jax
import jax.numpy as jnp
import numpy as np
from jax.experimental import pallas as pl
from jax.experimental.pallas import tpu as pltpu

BS, D, H, W = 1, 128, 84, 84
WS, TOPK, NHEAD = 7, 8, 8
M = H // WS
N = W // WS
NW = M * N
WS2 = WS * WS
DIM = D // NHEAD
WSP = 56
SLENP = TOPK * WSP + NW
QSP = NHEAD * WSP
NPADK = TOPK * (WSP - WS2)


PROJ_B = 48


def _proj_kernel(x_ref, s_ref, wq_ref, wk_ref, wv_ref,
                 q_ref, k_ref, v_ref, qm_ref, km_ref, vm_ref):
    xw = x_ref[...].reshape(PROJ_B * WSP, D)
    sw = s_ref[...].reshape(PROJ_B * WSP, D)
    nt = (((1,), (1,)), ((), ()))
    rcp = jnp.float32(1.0 / WS2)
    q = jax.lax.dot_general(xw, wq_ref[...], nt,
                            preferred_element_type=jnp.float32)
    k = jax.lax.dot_general(sw, wk_ref[...], nt,
                            preferred_element_type=jnp.float32)
    v = jax.lax.dot_general(sw, wv_ref[...], nt,
                            preferred_element_type=jnp.float32)
    q = q.reshape(PROJ_B, WSP, D)
    k = k.reshape(PROJ_B, WSP, D)
    v = v.reshape(PROJ_B, WSP, D)
    q_ref[...] = q
    k_ref[...] = k
    v_ref[...] = v
    qm_ref[...] = jnp.sum(q, axis=1, keepdims=True) * rcp
    km_ref[...] = jnp.sum(k, axis=1, keepdims=True) * rcp
    vm_ref[...] = jnp.sum(v, axis=1, keepdims=True) * rcp


def _topk_kernel(qm_ref, km_ref, idx_ref):
    qm = qm_ref[...].reshape(NW, D)
    km = km_ref[...].reshape(NW, D)
    sim = jax.lax.dot_general(qm, km, (((1,), (1,)), ((), ())),
                              preferred_element_type=jnp.float32)
    col = jax.lax.broadcasted_iota(jnp.int32, (NW, NW), 1)
    neg = jnp.float32(-jnp.inf)
    for j in range(TOPK):
        mx = jnp.max(sim, axis=1, keepdims=True)
        hit = sim >= mx
        cand = jnp.where(hit, col, NW)
        sel = jnp.min(cand, axis=1, keepdims=True)
        idx_ref[:, j:j + 1] = sel
        sim = jnp.where(col == sel, neg, sim)


ATTN_B = 16


def _attn_kernel(idx_ref, q_ref, k_ref, v_ref, km_ref, vm_ref, o_ref):
    i = pl.program_id(0)
    lane = jax.lax.broadcasted_iota(jnp.int32, (NHEAD, 1, D), 2)
    head = jax.lax.broadcasted_iota(jnp.int32, (NHEAD, 1, D), 0)
    mask = (lane // DIM == head).astype(jnp.float32)
    qmask = mask * jnp.float32(1.0 / np.sqrt(DIM))
    kmr = km_ref[...].reshape(NW, D)
    vmr = vm_ref[...].reshape(NW, D)
    for b in range(ATTN_B):
        w = i * ATTN_B + b
        qw = q_ref[b]
        qs = (qw[None, :, :] * qmask).reshape(QSP, D)
        parts_k = [k_ref[idx_ref[w, j]] for j in range(TOPK)]
        parts_v = [v_ref[idx_ref[w, j]] for j in range(TOPK)]
        kcat = jnp.concatenate(parts_k + [kmr], axis=0)
        vcat = jnp.concatenate(parts_v + [vmr], axis=0)

        s = jax.lax.dot_general(qs, kcat, (((1,), (1,)), ((), ())),
                                preferred_element_type=jnp.float32)
        e = jnp.exp(s)
        denom = jnp.sum(e, axis=1, keepdims=True) - jnp.float32(NPADK)
        msg = jax.lax.dot_general(e, vcat, (((1,), (0,)), ((), ())),
                                  preferred_element_type=jnp.float32)
        r = (1.0 / denom).reshape(NHEAD, WSP, 1)
        msg = jnp.sum(msg.reshape(NHEAD, WSP, D) * r * mask, axis=0)
        o_ref[b] = msg[:WS2]


def _layer_norm(v, g, b, eps=1e-5):
    mu = jnp.mean(v, axis=-1, keepdims=True)
    var = jnp.mean((v - mu) ** 2, axis=-1, keepdims=True)
    return (v - mu) / jnp.sqrt(var + eps) * g + b


def _ffn_kernel(xt_ref, msg_ref, wm_ref, fc1a_ref, fc1b_ref, fc1bias_ref,
                dw_ref, dwb_ref, fc2_ref, fc2b_ref, n1w_ref, n1b_ref,
                n2w_ref, n2b_ref, out_ref, pad_ref):
    nt = (((1,), (1,)), ((), ()))
    xt = xt_ref[...]
    merged = jax.lax.dot_general(msg_ref[...], wm_ref[...], nt,
                                 preferred_element_type=jnp.float32)
    merged = _layer_norm(merged, n1w_ref[0], n1b_ref[0])
    y = (jax.lax.dot_general(xt, fc1a_ref[...], nt,
                             preferred_element_type=jnp.float32)
         + jax.lax.dot_general(merged, fc1b_ref[...], nt,
                               preferred_element_type=jnp.float32)
         + fc1bias_ref[0])
    y = jnp.maximum(y, 0.0)
    pad_ref[...] = jnp.zeros_like(pad_ref)
    pad_ref[1:H + 1, 1:W + 1, :] = y.reshape(H, W, 2 * D)
    acc = jnp.zeros((H, W, 2 * D), jnp.float32)
    for di in range(3):
        for dj in range(3):
            acc = acc + pad_ref[di:di + H, dj:dj + W, :] * dw_ref[di * 3 + dj]
    y2 = acc.reshape(H * W, 2 * D) + dwb_ref[0]
    y2 = 0.5 * y2 * (1.0 + jax.lax.erf(y2 * jnp.float32(1.0 / np.sqrt(2.0))))
    z = jax.lax.dot_general(y2, fc2_ref[...], nt,
                            preferred_element_type=jnp.float32) + fc2b_ref[0]
    out_ref[...] = _layer_norm(z, n2w_ref[0], n2b_ref[0]) + xt


@jax.jit
def kernel(x, source, Wq, Wk, Wv, Wm, fc1_w, fc1_b, dw_w, dw_b, fc2_w, fc2_b,
           n1_w, n1_b, n2_w, n2_b):
    f32 = jnp.float32
    xt = jnp.transpose(x, (0, 2, 3, 1)).reshape(H * W, D)
    xv = jnp.pad(
        jnp.transpose(x.reshape(D, M, WS, N, WS), (1, 3, 2, 4, 0)
                      ).reshape(NW, WS2, D),
        ((0, 0), (0, WSP - WS2), (0, 0)))
    sv = jnp.pad(
        jnp.transpose(source.reshape(D, M, WS, N, WS), (1, 3, 2, 4, 0)
                      ).reshape(NW, WS2, D),
        ((0, 0), (0, WSP - WS2), (0, 0)))

    win_in = pl.BlockSpec((PROJ_B, WSP, D), lambda i: (i, 0, 0))
    full_w = pl.BlockSpec((D, D), lambda i: (0, 0))
    q, k, v, qm, km, vm = pl.pallas_call(
        _proj_kernel,
        grid=(NW // PROJ_B,),
        in_specs=[win_in, win_in, full_w, full_w, full_w],
        out_specs=[pl.BlockSpec((PROJ_B, WSP, D), lambda i: (i, 0, 0)),
                   pl.BlockSpec((PROJ_B, WSP, D), lambda i: (i, 0, 0)),
                   pl.BlockSpec((PROJ_B, WSP, D), lambda i: (i, 0, 0)),
                   pl.BlockSpec((PROJ_B, 1, D), lambda i: (i, 0, 0)),
                   pl.BlockSpec((PROJ_B, 1, D), lambda i: (i, 0, 0)),
                   pl.BlockSpec((PROJ_B, 1, D), lambda i: (i, 0, 0))],
        out_shape=[jax.ShapeDtypeStruct((NW, WSP, D), f32)] * 3
                  + [jax.ShapeDtypeStruct((NW, 1, D), f32)] * 3,
    )(xv, sv, Wq, Wk, Wv)

    idx = pl.pallas_call(
        _topk_kernel,
        in_specs=[pl.BlockSpec((NW, 1, D), lambda: (0, 0, 0)),
                  pl.BlockSpec((NW, 1, D), lambda: (0, 0, 0))],
        out_specs=pl.BlockSpec((NW, TOPK), lambda: (0, 0)),
        out_shape=jax.ShapeDtypeStruct((NW, TOPK), jnp.int32),
    )(qm, km)

    msg = pl.pallas_call(
        _attn_kernel,
        grid=(NW // ATTN_B,),
        in_specs=[pl.BlockSpec(memory_space=pltpu.SMEM),
                  pl.BlockSpec((ATTN_B, WSP, D), lambda i: (i, 0, 0)),
                  pl.BlockSpec((NW, WSP, D), lambda i: (0, 0, 0)),
                  pl.BlockSpec((NW, WSP, D), lambda i: (0, 0, 0)),
                  pl.BlockSpec((NW, 1, D), lambda i: (0, 0, 0)),
                  pl.BlockSpec((NW, 1, D), lambda i: (0, 0, 0))],
        out_specs=pl.BlockSpec((ATTN_B, WS2, D), lambda i: (i, 0, 0)),
        out_shape=jax.ShapeDtypeStruct((NW, WS2, D), f32),
    )(idx, q, k, v, km, vm)
    msg = msg.reshape(H * W, D)

    fc1a = fc1_w[:, :D]
    fc1b = fc1_w[:, D:]
    dwf = jnp.transpose(dw_w[:, 0].reshape(2 * D, 9), (1, 0))
    out2d = pl.pallas_call(
        _ffn_kernel,
        in_specs=[pl.BlockSpec((H * W, D), lambda: (0, 0)),
                  pl.BlockSpec((H * W, D), lambda: (0, 0)),
                  pl.BlockSpec((D, D), lambda: (0, 0)),
                  pl.BlockSpec((2 * D, D), lambda: (0, 0)),
                  pl.BlockSpec((2 * D, D), lambda: (0, 0)),
                  pl.BlockSpec((1, 2 * D), lambda: (0, 0)),
                  pl.BlockSpec((9, 2 * D), lambda: (0, 0)),
                  pl.BlockSpec((1, 2 * D), lambda: (0, 0)),
                  pl.BlockSpec((D, 2 * D), lambda: (0, 0)),
                  pl.BlockSpec((1, D), lambda: (0, 0)),
                  pl.BlockSpec((1, D), lambda: (0, 0)),
                  pl.BlockSpec((1, D), lambda: (0, 0)),
                  pl.BlockSpec((1, D), lambda: (0, 0)),
                  pl.BlockSpec((1, D), lambda: (0, 0))],
        out_specs=pl.BlockSpec((H * W, D), lambda: (0, 0)),
        out_shape=jax.ShapeDtypeStruct((H * W, D), f32),
        scratch_shapes=[pltpu.VMEM((H + 2, W + 2, 2 * D), f32)],
    )(xt, msg, Wm, fc1a, fc1b, fc1_b[None, :], dwf, dw_b[None, :], fc2_w,
      fc2_b[None, :], n1_w[None, :], n1_b[None, :], n2_w[None, :],
      n2_b[None, :])

    out = jnp.transpose(out2d.reshape(1, H, W, D), (0, 3, 1, 2))
    return out


# speedup vs baseline: 2.0035x; 1.0052x over previous
import jax
import jax.numpy as jnp
import numpy as np
from jax.experimental import pallas as pl
from jax.experimental.pallas import tpu as pltpu

BS, D, H, W = 1, 128, 84, 84
WS, TOPK, NHEAD = 7, 8, 8
M = H // WS
N = W // WS
NW = M * N
WS2 = WS * WS
DIM = D // NHEAD
WSP = 56
SLENP = TOPK * WSP + NW
QSP = NHEAD * WSP
NPADK = TOPK * (WSP - WS2)


PROJ_B = 144


def _proj_kernel(x_ref, s_ref, wq_ref, wk_ref, wv_ref,
                 q_ref, k_ref, v_ref, qm_ref, km_ref, vm_ref):
    xw = x_ref[...].reshape(PROJ_B * WSP, D)
    sw = s_ref[...].reshape(PROJ_B * WSP, D)
    nt = (((1,), (1,)), ((), ()))
    rcp = jnp.float32(1.0 / WS2)
    q = jax.lax.dot_general(xw, wq_ref[...], nt,
                            preferred_element_type=jnp.float32)
    k = jax.lax.dot_general(sw, wk_ref[...], nt,
                            preferred_element_type=jnp.float32)
    v = jax.lax.dot_general(sw, wv_ref[...], nt,
                            preferred_element_type=jnp.float32)
    q = q.reshape(PROJ_B, WSP, D)
    k = k.reshape(PROJ_B, WSP, D)
    v = v.reshape(PROJ_B, WSP, D)
    q_ref[...] = q
    k_ref[...] = k
    v_ref[...] = v
    qm_ref[...] = jnp.sum(q, axis=1, keepdims=True) * rcp
    km_ref[...] = jnp.sum(k, axis=1, keepdims=True) * rcp
    vm_ref[...] = jnp.sum(v, axis=1, keepdims=True) * rcp


def _topk_kernel(qm_ref, km_ref, idx_ref):
    qm = qm_ref[...].reshape(NW, D)
    km = km_ref[...].reshape(NW, D)
    sim = jax.lax.dot_general(qm, km, (((1,), (1,)), ((), ())),
                              preferred_element_type=jnp.float32)
    col = jax.lax.broadcasted_iota(jnp.int32, (NW, NW), 1)
    neg = jnp.float32(-jnp.inf)
    for j in range(TOPK):
        mx = jnp.max(sim, axis=1, keepdims=True)
        hit = sim >= mx
        cand = jnp.where(hit, col, NW)
        sel = jnp.min(cand, axis=1, keepdims=True)
        idx_ref[:, j:j + 1] = sel
        sim = jnp.where(col == sel, neg, sim)


ATTN_B = 24


def _attn_kernel(idx_ref, q_ref, k_ref, v_ref, km_ref, vm_ref, o_ref):
    i = pl.program_id(0)
    lane = jax.lax.broadcasted_iota(jnp.int32, (NHEAD, 1, D), 2)
    head = jax.lax.broadcasted_iota(jnp.int32, (NHEAD, 1, D), 0)
    mask = (lane // DIM == head).astype(jnp.float32)
    qmask = mask * jnp.float32(1.0 / np.sqrt(DIM))
    kmr = km_ref[...].reshape(NW, D)
    vmr = vm_ref[...].reshape(NW, D)
    for b in range(ATTN_B):
        w = i * ATTN_B + b
        qw = q_ref[b]
        qs = (qw[None, :, :] * qmask).reshape(QSP, D)
        parts_k = [k_ref[idx_ref[w, j]] for j in range(TOPK)]
        parts_v = [v_ref[idx_ref[w, j]] for j in range(TOPK)]
        kcat = jnp.concatenate(parts_k + [kmr], axis=0)
        vcat = jnp.concatenate(parts_v + [vmr], axis=0)

        s = jax.lax.dot_general(qs, kcat, (((1,), (1,)), ((), ())),
                                preferred_element_type=jnp.float32)
        e = jnp.exp(s)
        denom = jnp.sum(e, axis=1, keepdims=True) - jnp.float32(NPADK)
        msg = jax.lax.dot_general(e, vcat, (((1,), (0,)), ((), ())),
                                  preferred_element_type=jnp.float32)
        r = (1.0 / denom).reshape(NHEAD, WSP, 1)
        msg = jnp.sum(msg.reshape(NHEAD, WSP, D) * r * mask, axis=0)
        o_ref[b] = msg[:WS2]


def _layer_norm(v, g, b, eps=1e-5):
    mu = jnp.mean(v, axis=-1, keepdims=True)
    var = jnp.mean((v - mu) ** 2, axis=-1, keepdims=True)
    return (v - mu) / jnp.sqrt(var + eps) * g + b


def _ffn_kernel(xt_ref, msg_ref, wm_ref, fc1a_ref, fc1b_ref, fc1bias_ref,
                dw_ref, dwb_ref, fc2_ref, fc2b_ref, n1w_ref, n1b_ref,
                n2w_ref, n2b_ref, out_ref, pad_ref):
    nt = (((1,), (1,)), ((), ()))
    xt = xt_ref[...]
    merged = jax.lax.dot_general(msg_ref[...], wm_ref[...], nt,
                                 preferred_element_type=jnp.float32)
    merged = _layer_norm(merged, n1w_ref[0], n1b_ref[0])
    y = (jax.lax.dot_general(xt, fc1a_ref[...], nt,
                             preferred_element_type=jnp.float32)
         + jax.lax.dot_general(merged, fc1b_ref[...], nt,
                               preferred_element_type=jnp.float32)
         + fc1bias_ref[0])
    y = jnp.maximum(y, 0.0)
    pad_ref[...] = jnp.zeros_like(pad_ref)
    pad_ref[1:H + 1, 1:W + 1, :] = y.reshape(H, W, 2 * D)
    acc = jnp.zeros((H, W, 2 * D), jnp.float32)
    for di in range(3):
        for dj in range(3):
            acc = acc + pad_ref[di:di + H, dj:dj + W, :] * dw_ref[di * 3 + dj]
    y2 = acc.reshape(H * W, 2 * D) + dwb_ref[0]
    y2 = 0.5 * y2 * (1.0 + jax.lax.erf(y2 * jnp.float32(1.0 / np.sqrt(2.0))))
    z = jax.lax.dot_general(y2, fc2_ref[...], nt,
                            preferred_element_type=jnp.float32) + fc2b_ref[0]
    out_ref[...] = _layer_norm(z, n2w_ref[0], n2b_ref[0]) + xt


@jax.jit
def kernel(x, source, Wq, Wk, Wv, Wm, fc1_w, fc1_b, dw_w, dw_b, fc2_w, fc2_b,
           n1_w, n1_b, n2_w, n2_b):
    f32 = jnp.float32
    xt = jnp.transpose(x, (0, 2, 3, 1)).reshape(H * W, D)
    xv = jnp.pad(
        jnp.transpose(x.reshape(D, M, WS, N, WS), (1, 3, 2, 4, 0)
                      ).reshape(NW, WS2, D),
        ((0, 0), (0, WSP - WS2), (0, 0)))
    sv = jnp.pad(
        jnp.transpose(source.reshape(D, M, WS, N, WS), (1, 3, 2, 4, 0)
                      ).reshape(NW, WS2, D),
        ((0, 0), (0, WSP - WS2), (0, 0)))

    win_in = pl.BlockSpec((PROJ_B, WSP, D), lambda i: (i, 0, 0))
    full_w = pl.BlockSpec((D, D), lambda i: (0, 0))
    q, k, v, qm, km, vm = pl.pallas_call(
        _proj_kernel,
        grid=(NW // PROJ_B,),
        in_specs=[win_in, win_in, full_w, full_w, full_w],
        out_specs=[pl.BlockSpec((PROJ_B, WSP, D), lambda i: (i, 0, 0)),
                   pl.BlockSpec((PROJ_B, WSP, D), lambda i: (i, 0, 0)),
                   pl.BlockSpec((PROJ_B, WSP, D), lambda i: (i, 0, 0)),
                   pl.BlockSpec((PROJ_B, 1, D), lambda i: (i, 0, 0)),
                   pl.BlockSpec((PROJ_B, 1, D), lambda i: (i, 0, 0)),
                   pl.BlockSpec((PROJ_B, 1, D), lambda i: (i, 0, 0))],
        out_shape=[jax.ShapeDtypeStruct((NW, WSP, D), f32)] * 3
                  + [jax.ShapeDtypeStruct((NW, 1, D), f32)] * 3,
    )(xv, sv, Wq, Wk, Wv)

    idx = pl.pallas_call(
        _topk_kernel,
        in_specs=[pl.BlockSpec((NW, 1, D), lambda: (0, 0, 0)),
                  pl.BlockSpec((NW, 1, D), lambda: (0, 0, 0))],
        out_specs=pl.BlockSpec((NW, TOPK), lambda: (0, 0)),
        out_shape=jax.ShapeDtypeStruct((NW, TOPK), jnp.int32),
    )(qm, km)

    msg = pl.pallas_call(
        _attn_kernel,
        grid=(NW // ATTN_B,),
        in_specs=[pl.BlockSpec(memory_space=pltpu.SMEM),
                  pl.BlockSpec((ATTN_B, WSP, D), lambda i: (i, 0, 0)),
                  pl.BlockSpec((NW, WSP, D), lambda i: (0, 0, 0)),
                  pl.BlockSpec((NW, WSP, D), lambda i: (0, 0, 0)),
                  pl.BlockSpec((NW, 1, D), lambda i: (0, 0, 0)),
                  pl.BlockSpec((NW, 1, D), lambda i: (0, 0, 0))],
        out_specs=pl.BlockSpec((ATTN_B, WS2, D), lambda i: (i, 0, 0)),
        out_shape=jax.ShapeDtypeStruct((NW, WS2, D), f32),
    )(idx, q, k, v, km, vm)
    msg = msg.reshape(H * W, D)

    fc1a = fc1_w[:, :D]
    fc1b = fc1_w[:, D:]
    dwf = jnp.transpose(dw_w[:, 0].reshape(2 * D, 9), (1, 0))
    out2d = pl.pallas_call(
        _ffn_kernel,
        in_specs=[pl.BlockSpec((H * W, D), lambda: (0, 0)),
                  pl.BlockSpec((H * W, D), lambda: (0, 0)),
                  pl.BlockSpec((D, D), lambda: (0, 0)),
                  pl.BlockSpec((2 * D, D), lambda: (0, 0)),
                  pl.BlockSpec((2 * D, D), lambda: (0, 0)),
                  pl.BlockSpec((1, 2 * D), lambda: (0, 0)),
                  pl.BlockSpec((9, 2 * D), lambda: (0, 0)),
                  pl.BlockSpec((1, 2 * D), lambda: (0, 0)),
                  pl.BlockSpec((D, 2 * D), lambda: (0, 0)),
                  pl.BlockSpec((1, D), lambda: (0, 0)),
                  pl.BlockSpec((1, D), lambda: (0, 0)),
                  pl.BlockSpec((1, D), lambda: (0, 0)),
                  pl.BlockSpec((1, D), lambda: (0, 0)),
                  pl.BlockSpec((1, D), lambda: (0, 0))],
        out_specs=pl.BlockSpec((H * W, D), lambda: (0, 0)),
        out_shape=jax.ShapeDtypeStruct((H * W, D), f32),
        scratch_shapes=[pltpu.VMEM((H + 2, W + 2, 2 * D), f32)],
    )(xt, msg, Wm, fc1a, fc1b, fc1_b[None, :], dwf, dw_b[None, :], fc2_w,
      fc2_b[None, :], n1_w[None, :], n1_b[None, :], n2_w[None, :],
      n2_b[None, :])

    out = jnp.transpose(out2d.reshape(1, H, W, D), (0, 3, 1, 2))
    return out


# speedup vs baseline: 2.0602x; 1.0283x over previous
import jax
import jax.numpy as jnp
import numpy as np
from jax.experimental import pallas as pl
from jax.experimental.pallas import tpu as pltpu

BS, D, H, W = 1, 128, 84, 84
WS, TOPK, NHEAD = 7, 8, 8
M = H // WS
N = W // WS
NW = M * N
WS2 = WS * WS
DIM = D // NHEAD
WSP = 56
SLENP = TOPK * WSP + NW
QSP = NHEAD * WSP
NPADK = TOPK * (WSP - WS2)


PROJ_B = 144


def _proj_kernel(x_ref, s_ref, wq_ref, wk_ref, wv_ref,
                 q_ref, k_ref, v_ref, km_ref, vm_ref, idx_ref):
    xw = x_ref[...].reshape(PROJ_B * WSP, D)
    sw = s_ref[...].reshape(PROJ_B * WSP, D)
    nt = (((1,), (1,)), ((), ()))
    rcp = jnp.float32(1.0 / WS2)
    q = jax.lax.dot_general(xw, wq_ref[...], nt,
                            preferred_element_type=jnp.float32)
    k = jax.lax.dot_general(sw, wk_ref[...], nt,
                            preferred_element_type=jnp.float32)
    v = jax.lax.dot_general(sw, wv_ref[...], nt,
                            preferred_element_type=jnp.float32)
    q = q.reshape(PROJ_B, WSP, D)
    k = k.reshape(PROJ_B, WSP, D)
    v = v.reshape(PROJ_B, WSP, D)
    q_ref[...] = q
    k_ref[...] = k
    v_ref[...] = v
    qm = jnp.sum(q, axis=1) * rcp
    km = jnp.sum(k, axis=1) * rcp
    km_ref[...] = km[:, None, :]
    vm_ref[...] = jnp.sum(v, axis=1, keepdims=True) * rcp
    sim = jax.lax.dot_general(qm, km, (((1,), (1,)), ((), ())),
                              preferred_element_type=jnp.float32)
    col = jax.lax.broadcasted_iota(jnp.int32, (NW, NW), 1)
    neg = jnp.float32(-jnp.inf)
    for j in range(TOPK):
        mx = jnp.max(sim, axis=1, keepdims=True)
        hit = sim >= mx
        cand = jnp.where(hit, col, NW)
        sel = jnp.min(cand, axis=1, keepdims=True)
        idx_ref[:, j:j + 1] = sel
        sim = jnp.where(col == sel, neg, sim)


ATTN_B = 48


def _attn_kernel(idx_ref, q_ref, k_ref, v_ref, km_ref, vm_ref, o_ref):
    i = pl.program_id(0)
    lane = jax.lax.broadcasted_iota(jnp.int32, (NHEAD, 1, D), 2)
    head = jax.lax.broadcasted_iota(jnp.int32, (NHEAD, 1, D), 0)
    mask = (lane // DIM == head).astype(jnp.float32)
    qmask = mask * jnp.float32(1.0 / np.sqrt(DIM))
    kmr = km_ref[...].reshape(NW, D)
    vmr = vm_ref[...].reshape(NW, D)
    for b in range(ATTN_B):
        w = i * ATTN_B + b
        qw = q_ref[b]
        qs = (qw[None, :, :] * qmask).reshape(QSP, D)
        parts_k = [k_ref[idx_ref[w, j]] for j in range(TOPK)]
        parts_v = [v_ref[idx_ref[w, j]] for j in range(TOPK)]
        kcat = jnp.concatenate(parts_k + [kmr], axis=0)
        vcat = jnp.concatenate(parts_v + [vmr], axis=0)

        s = jax.lax.dot_general(qs, kcat, (((1,), (1,)), ((), ())),
                                preferred_element_type=jnp.float32)
        e = jnp.exp(s)
        denom = jnp.sum(e, axis=1, keepdims=True) - jnp.float32(NPADK)
        msg = jax.lax.dot_general(e, vcat, (((1,), (0,)), ((), ())),
                                  preferred_element_type=jnp.float32)
        r = (1.0 / denom).reshape(NHEAD, WSP, 1)
        msg = jnp.sum(msg.reshape(NHEAD, WSP, D) * r * mask, axis=0)
        o_ref[b] = msg[:WS2]


def _layer_norm(v, g, b, eps=1e-5):
    mu = jnp.mean(v, axis=-1, keepdims=True)
    var = jnp.mean((v - mu) ** 2, axis=-1, keepdims=True)
    return (v - mu) / jnp.sqrt(var + eps) * g + b


def _ffn_kernel(xt_ref, msg_ref, wm_ref, fc1a_ref, fc1b_ref, fc1bias_ref,
                dw_ref, dwb_ref, fc2_ref, fc2b_ref, n1w_ref, n1b_ref,
                n2w_ref, n2b_ref, out_ref, pad_ref):
    nt = (((1,), (1,)), ((), ()))
    xt = xt_ref[...]
    merged = jax.lax.dot_general(msg_ref[...], wm_ref[...], nt,
                                 preferred_element_type=jnp.float32)
    merged = _layer_norm(merged, n1w_ref[0], n1b_ref[0])
    y = (jax.lax.dot_general(xt, fc1a_ref[...], nt,
                             preferred_element_type=jnp.float32)
         + jax.lax.dot_general(merged, fc1b_ref[...], nt,
                               preferred_element_type=jnp.float32)
         + fc1bias_ref[0])
    y = jnp.maximum(y, 0.0)
    pad_ref[...] = jnp.zeros_like(pad_ref)
    pad_ref[1:H + 1, 1:W + 1, :] = y.reshape(H, W, 2 * D)
    acc = jnp.zeros((H, W, 2 * D), jnp.float32)
    for di in range(3):
        for dj in range(3):
            acc = acc + pad_ref[di:di + H, dj:dj + W, :] * dw_ref[di * 3 + dj]
    y2 = acc.reshape(H * W, 2 * D) + dwb_ref[0]
    y2 = 0.5 * y2 * (1.0 + jax.lax.erf(y2 * jnp.float32(1.0 / np.sqrt(2.0))))
    z = jax.lax.dot_general(y2, fc2_ref[...], nt,
                            preferred_element_type=jnp.float32) + fc2b_ref[0]
    out_ref[...] = _layer_norm(z, n2w_ref[0], n2b_ref[0]) + xt


@jax.jit
def kernel(x, source, Wq, Wk, Wv, Wm, fc1_w, fc1_b, dw_w, dw_b, fc2_w, fc2_b,
           n1_w, n1_b, n2_w, n2_b):
    f32 = jnp.float32
    xt = jnp.transpose(x, (0, 2, 3, 1)).reshape(H * W, D)
    xv = jnp.pad(
        jnp.transpose(x.reshape(D, M, WS, N, WS), (1, 3, 2, 4, 0)
                      ).reshape(NW, WS2, D),
        ((0, 0), (0, WSP - WS2), (0, 0)))
    sv = jnp.pad(
        jnp.transpose(source.reshape(D, M, WS, N, WS), (1, 3, 2, 4, 0)
                      ).reshape(NW, WS2, D),
        ((0, 0), (0, WSP - WS2), (0, 0)))

    win_in = pl.BlockSpec((PROJ_B, WSP, D), lambda i: (i, 0, 0))
    full_w = pl.BlockSpec((D, D), lambda i: (0, 0))
    q, k, v, km, vm, idx = pl.pallas_call(
        _proj_kernel,
        grid=(NW // PROJ_B,),
        in_specs=[win_in, win_in, full_w, full_w, full_w],
        out_specs=[pl.BlockSpec((PROJ_B, WSP, D), lambda i: (i, 0, 0)),
                   pl.BlockSpec((PROJ_B, WSP, D), lambda i: (i, 0, 0)),
                   pl.BlockSpec((PROJ_B, WSP, D), lambda i: (i, 0, 0)),
                   pl.BlockSpec((PROJ_B, 1, D), lambda i: (i, 0, 0)),
                   pl.BlockSpec((PROJ_B, 1, D), lambda i: (i, 0, 0)),
                   pl.BlockSpec((NW, TOPK), lambda i: (0, 0))],
        out_shape=[jax.ShapeDtypeStruct((NW, WSP, D), f32)] * 3
                  + [jax.ShapeDtypeStruct((NW, 1, D), f32)] * 2
                  + [jax.ShapeDtypeStruct((NW, TOPK), jnp.int32)],
    )(xv, sv, Wq, Wk, Wv)

    msg = pl.pallas_call(
        _attn_kernel,
        grid=(NW // ATTN_B,),
        in_specs=[pl.BlockSpec(memory_space=pltpu.SMEM),
                  pl.BlockSpec((ATTN_B, WSP, D), lambda i: (i, 0, 0)),
                  pl.BlockSpec((NW, WSP, D), lambda i: (0, 0, 0)),
                  pl.BlockSpec((NW, WSP, D), lambda i: (0, 0, 0)),
                  pl.BlockSpec((NW, 1, D), lambda i: (0, 0, 0)),
                  pl.BlockSpec((NW, 1, D), lambda i: (0, 0, 0))],
        out_specs=pl.BlockSpec((ATTN_B, WS2, D), lambda i: (i, 0, 0)),
        out_shape=jax.ShapeDtypeStruct((NW, WS2, D), f32),
    )(idx, q, k, v, km, vm)
    msg = msg.reshape(H * W, D)

    fc1a = fc1_w[:, :D]
    fc1b = fc1_w[:, D:]
    dwf = jnp.transpose(dw_w[:, 0].reshape(2 * D, 9), (1, 0))
    out2d = pl.pallas_call(
        _ffn_kernel,
        in_specs=[pl.BlockSpec((H * W, D), lambda: (0, 0)),
                  pl.BlockSpec((H * W, D), lambda: (0, 0)),
                  pl.BlockSpec((D, D), lambda: (0, 0)),
                  pl.BlockSpec((2 * D, D), lambda: (0, 0)),
                  pl.BlockSpec((2 * D, D), lambda: (0, 0)),
                  pl.BlockSpec((1, 2 * D), lambda: (0, 0)),
                  pl.BlockSpec((9, 2 * D), lambda: (0, 0)),
                  pl.BlockSpec((1, 2 * D), lambda: (0, 0)),
                  pl.BlockSpec((D, 2 * D), lambda: (0, 0)),
                  pl.BlockSpec((1, D), lambda: (0, 0)),
                  pl.BlockSpec((1, D), lambda: (0, 0)),
                  pl.BlockSpec((1, D), lambda: (0, 0)),
                  pl.BlockSpec((1, D), lambda: (0, 0)),
                  pl.BlockSpec((1, D), lambda: (0, 0))],
        out_specs=pl.BlockSpec((H * W, D), lambda: (0, 0)),
        out_shape=jax.ShapeDtypeStruct((H * W, D), f32),
        scratch_shapes=[pltpu.VMEM((H + 2, W + 2, 2 * D), f32)],
    )(xt, msg, Wm, fc1a, fc1b, fc1_b[None, :], dwf, dw_b[None, :], fc2_w,
      fc2_b[None, :], n1_w[None, :], n1_b[None, :], n2_w[None, :],
      n2_b[None, :])

    out = jnp.transpose(out2d.reshape(1, H, W, D), (0, 3, 1, 2))
    return out
